# Initial kernel scaffold; baseline (speedup 1.0000x reference)
import jax
import jax.numpy as jnp
from jax.experimental import pallas as pl


def kernel(x, pos_idx, edge_index, edge_type, pos_emb, bases1, comp1, root1, bias1, bases2, comp2, root2, bias2, W1, b1, W2, b2):
    raise NotImplementedError("write your pallas kernel here")


# speedup vs baseline: 2.1033x; 2.1033x over previous
import functools
import jax
import jax.numpy as jnp
from jax.experimental import pallas as pl

N = 10000
E = 160000
R = 8
NB = 4
D = 240
XD = 200
ED = 40
V = 64
H1 = 128

_SLOPE = 0.01


def _mm_kernel(a_ref, b_ref, o_ref):
    o_ref[...] = jnp.dot(a_ref[...], b_ref[...],
                         preferred_element_type=jnp.float32)


def _mm(a, b, bn=1000):
    n, k = a.shape
    k2, m = b.shape
    assert k == k2 and n % bn == 0
    return pl.pallas_call(
        _mm_kernel,
        grid=(n // bn,),
        in_specs=[
            pl.BlockSpec((bn, k), lambda i: (i, 0)),
            pl.BlockSpec((k, m), lambda i: (0, 0)),
        ],
        out_specs=pl.BlockSpec((bn, m), lambda i: (i, 0)),
        out_shape=jax.ShapeDtypeStruct((n, m), jnp.float32),
    )(a, b)


def kernel(x, pos_idx, edge_index, edge_type, pos_emb, bases1, comp1, root1,
           bias1, bases2, comp2, root2, bias2, W1, b1, W2, b2):
    src = edge_index[0]
    dst = edge_index[1]
    et = edge_type

    emb = jnp.take(pos_emb, pos_idx, axis=0)
    h0 = jnp.concatenate([x.astype(jnp.float32), emb], axis=-1)

    idxs = jnp.arange(N)
    concept_index = jnp.max(jnp.where(pos_idx == 0, idxs, -1))
    sentence_index = jnp.max(jnp.where(pos_idx == 1, idxs, -1))

    seg = dst * R + et
    cnt = jax.ops.segment_sum(jnp.ones((E,), jnp.float32), seg,
                              num_segments=N * R)
    inv = 1.0 / jnp.maximum(cnt, 1.0)
    w = inv[seg]

    Wcat1 = jnp.einsum('rb,bdf->drf', comp1, bases1).reshape(D, R * D)
    Wcat1 = jnp.concatenate([Wcat1, root1], axis=1)
    Wcat2 = jnp.einsum('rb,bdf->drf', comp2, bases2).reshape(D, R * D)
    Wcat2 = jnp.concatenate([Wcat2, root2], axis=1)

    def layer(h, Wcat, bias):
        hr = _mm(h, Wcat)
        selfp = hr[:, R * D:]
        rows = hr[:, :R * D].reshape(N * R, D)
        gathered = jnp.take(rows, src * R + et, axis=0)
        agg = jax.ops.segment_sum(w[:, None] * gathered, dst,
                                  num_segments=N)
        return agg + selfp + bias[None, :]

    h1 = jax.nn.leaky_relu(layer(h0, Wcat1, bias1), _SLOPE)
    h2 = layer(h1, Wcat2, bias2)

    hs = h2[sentence_index]
    hc = h2[concept_index]
    h_cat = jnp.concatenate([jnp.abs(hs - hc), hs * hc], axis=-1)
    hid = jax.nn.leaky_relu(h_cat @ W1 + b1, _SLOPE)
    return hid @ W2 + b2


# speedup vs baseline: 12.1934x; 5.7974x over previous
import jax
import jax.numpy as jnp
from jax import lax
from jax.experimental import pallas as pl
from jax.experimental.pallas import tpu as pltpu
from jax.experimental.pallas import tpu_sc as plsc

N = 10000
E = 160000
R = 8
NB = 4
D = 240
XD = 200
ED = 40
V = 64
H1 = 128

_SLOPE = 0.01

NC = 2
NS = 16
L = 16

_MESH = plsc.VectorSubcoreMesh(core_axis_name="c", subcore_axis_name="s")
_PARAMS = pltpu.CompilerParams(needs_layout_passes=False)

EC = E // NS
EW = E // (NC * NS)
NW = NC * NS
SEGS = N * R
SEG_T = SEGS // NS
HALF = N // 2
CH = 80
CAP = 5120
NCHB = CAP // CH
SENT = 1 << 30


def _k0_body(dst_hbm, et_hbm, src_hbm, w_hbm, idxg_hbm, cnt_hbm,
             bufA, bufB, bufC, bufD, segS, cntL, cnt_sh):
    cid = lax.axis_index("c")
    sid = lax.axis_index("s")

    def z(i, _):
        bufD[pl.ds(i * L, L)] = jnp.zeros((L,), jnp.float32)
        return 0
    lax.fori_loop(0, (SEG_T + L - 1) // L, z, 0)
    pltpu.sync_copy(bufD.at[pl.ds(0, SEG_T)],
                    cnt_sh.at[pl.ds(sid * SEG_T, SEG_T)])
    plsc.subcore_barrier()

    base2 = sid * EC
    pltpu.sync_copy(dst_hbm.at[pl.ds(base2, EC)], bufA)
    pltpu.sync_copy(et_hbm.at[pl.ds(base2, EC)], bufB)
    ones = jnp.ones((L,), jnp.float32)

    def mkseg(i, _):
        s = pl.ds(i * L, L)
        bufC[s] = bufA[s] * R + bufB[s]
        bufD[s] = ones
        return 0
    lax.fori_loop(0, EC // L, mkseg, 0)

    def scat(j, _):
        row = j % 8

        def mv(k, _):
            s = pl.ds(k * L, L)
            segS[row, s] = bufC[pl.ds(j * 80 + k * L, L)]
            return 0
        lax.fori_loop(0, 80 // L, mv, 0)
        pltpu.sync_copy(bufD.at[pl.ds(j * 80, 80)], cnt_sh.at[segS.at[row]],
                        add=True)
        return 0
    lax.fori_loop(0, EC // 80, scat, 0)
    plsc.subcore_barrier()

    pltpu.sync_copy(cnt_sh, cntL)
    wid = sid * NC + cid
    base3 = wid * EW
    pltpu.sync_copy(dst_hbm.at[pl.ds(base3, EW)], bufA.at[pl.ds(0, EW)])
    pltpu.sync_copy(et_hbm.at[pl.ds(base3, EW)], bufB.at[pl.ds(0, EW)])
    pltpu.sync_copy(src_hbm.at[pl.ds(base3, EW)], bufC.at[pl.ds(0, EW)])

    def per(i, _):
        s = pl.ds(i * L, L)
        dv = bufA[s]
        ev = bufB[s]
        sv = bufC[s]
        segv = dv * R + ev
        cv = plsc.load_gather(cntL, [segv])
        bufD[s] = 1.0 / jnp.maximum(cv, 1.0)
        bufC[s] = ev * N + sv
        return 0
    lax.fori_loop(0, (EW + L - 1) // L, per, 0)
    pltpu.sync_copy(bufD.at[pl.ds(0, EW)], w_hbm.at[pl.ds(base3, EW)])
    pltpu.sync_copy(bufC.at[pl.ds(0, EW)], idxg_hbm.at[pl.ds(base3, EW)])

    @pl.when(cid == 0)
    def _():
        pltpu.sync_copy(cntL.at[pl.ds(sid * SEG_T, SEG_T)],
                        cnt_hbm.at[pl.ds(sid * SEG_T, SEG_T)])


def _k0(dst, et, src):
    return pl.kernel(
        _k0_body,
        out_type=[jax.ShapeDtypeStruct((E,), jnp.float32),
                  jax.ShapeDtypeStruct((E,), jnp.int32),
                  jax.ShapeDtypeStruct((SEGS,), jnp.float32)],
        mesh=_MESH,
        compiler_params=_PARAMS,
        scratch_types=[
            pltpu.VMEM((EC,), jnp.int32),
            pltpu.VMEM((EC,), jnp.int32),
            pltpu.VMEM((EC,), jnp.int32),
            pltpu.VMEM((EC,), jnp.float32),
            pltpu.VMEM((8, 80), jnp.int32),
            pltpu.VMEM((SEGS,), jnp.float32),
            pltpu.VMEM_SHARED((SEGS,), jnp.float32),
        ],
    )(dst, et, src)


BN = 1000
NBLK = N // BN


def _k1_body(x_ref, pos_ref, pe_ref, basesp_ref, comp_ref, rootp_ref,
             biasp_ref, hr_ref, selfp_ref, cs_ref):
    i = pl.program_id(0)
    rj = pl.program_id(2)
    pos = pos_ref[0]
    oh = (pos.reshape(BN, 1) ==
          lax.broadcasted_iota(jnp.int32, (1, V), 1)).astype(jnp.float32)
    emb = jnp.dot(oh, pe_ref[...], preferred_element_type=jnp.float32)
    h = jnp.concatenate([x_ref[...], emb], axis=1)

    @pl.when(rj < R)
    def _():
        wb = comp_ref[rj, 0] * basesp_ref[0, 0]
        for b in range(1, NB):
            wb = wb + comp_ref[rj, b] * basesp_ref[0, b]
        hr_ref[0] = jnp.dot(h, wb, preferred_element_type=jnp.float32)

    @pl.when(rj == R)
    def _():
        selfp_ref[0] = (jnp.dot(h, rootp_ref[0],
                                preferred_element_type=jnp.float32)
                        + biasp_ref[0])

    @pl.when((pl.program_id(1) == 0) & (rj == 0))
    def _():
        idxs = i * BN + lax.broadcasted_iota(jnp.int32, (1, BN), 1)
        m0 = jnp.max(jnp.where(pos == 0, idxs, -1))
        m1 = jnp.max(jnp.where(pos == 1, idxs, -1))
        io = lax.broadcasted_iota(jnp.int32, (1, 16), 1)
        upd = jnp.where(io == 0, m0, jnp.where(io == 1, m1, -1))
        prev = jnp.where(i == 0, jnp.full((1, 16), -1, jnp.int32),
                         cs_ref[...])
        cs_ref[...] = jnp.maximum(prev, upd)


def _k1(x, pos3, pos_emb, basesp, comp, rootp, biasp):
    return pl.pallas_call(
        _k1_body,
        grid=(NBLK, NC, R + 1),
        in_specs=[
            pl.BlockSpec((BN, XD), lambda i, c, rj: (i, 0)),
            pl.BlockSpec((1, 1, BN), lambda i, c, rj: (i, 0, 0)),
            pl.BlockSpec((V, ED), lambda i, c, rj: (0, 0)),
            pl.BlockSpec((1, NB, D, 128), lambda i, c, rj: (c, 0, 0, 0)),
            pl.BlockSpec(memory_space=pltpu.SMEM),
            pl.BlockSpec((1, D, 128), lambda i, c, rj: (c, 0, 0)),
            pl.BlockSpec((1, 1, 128), lambda i, c, rj: (c, 0, 0)),
        ],
        out_specs=[
            pl.BlockSpec((1, BN, 128),
                         lambda i, c, rj: (c, jnp.minimum(rj, R - 1) * NBLK + i, 0)),
            pl.BlockSpec((1, BN, 128), lambda i, c, rj: (c, i, 0)),
            pl.BlockSpec((1, 16), lambda i, c, rj: (0, 0)),
        ],
        out_shape=[
            jax.ShapeDtypeStruct((NC, R * N, 128), jnp.float32),
            jax.ShapeDtypeStruct((NC, N, 128), jnp.float32),
            jax.ShapeDtypeStruct((1, 16), jnp.int32),
        ],
    )(x, pos3, pos_emb, basesp, comp, rootp, biasp)


CH = 80
NCHK = EC // CH
CHD = 80
NDR = N // CHD


def _k2_body(hr_ref, idx_hbm, dst_ref, w_hbm, agg_ref,
             I1, D2, W1b, rows, agg_sh, sem):
    cid = lax.axis_index("c")
    sid = lax.axis_index("s")

    pltpu.sync_copy(idx_hbm.at[pl.ds(sid * EC, EC)], I1)
    pltpu.sync_copy(dst_ref.at[sid], D2)
    pltpu.sync_copy(w_hbm.at[pl.ds(sid * EC, EC)], W1b)

    def zr(k, _):
        def zc(m, _):
            rows[k, pl.ds(m * L, L)] = jnp.zeros((L,), jnp.float32)
            return 0
        lax.fori_loop(0, 128 // L, zc, 0)
        return 0
    lax.fori_loop(0, CHD, zr, 0)
    for t in range((NDR + NS - 1) // NS):
        ci = sid + t * NS

        @pl.when(ci < NDR)
        def _():
            pltpu.sync_copy(rows, agg_sh.at[pl.ds(ci * CHD, CHD)])
    plsc.subcore_barrier()

    def chunk(j, _):
        for c in range(NC):
            @pl.when(cid == c)
            def _(c=c):
                pltpu.async_copy(hr_ref.at[c].at[I1.at[pl.ds(j * CH, CH)]],
                                 rows, sem).wait()

        def rowblk(kb, _):
            wv16 = W1b[pl.ds(j * CH + kb * L, L)]
            for l in range(L):
                wv = wv16[l]
                k = kb * L + l
                for m in range(128 // L):
                    s = pl.ds(m * L, L)
                    rows[k, s] = rows[k, s] * wv
            return 0
        lax.fori_loop(0, CH // L, rowblk, 0)
        pltpu.sync_copy(rows, agg_sh.at[D2.at[j]], add=True)
        return 0
    lax.fori_loop(0, NCHK, chunk, 0)
    plsc.subcore_barrier()

    for t in range((NDR + NS - 1) // NS):
        ci = sid + t * NS

        @pl.when(ci < NDR)
        def _():
            base = ci * CHD
            pltpu.sync_copy(agg_sh.at[pl.ds(base, CHD)], rows)
            for c in range(NC):
                @pl.when(cid == c)
                def _(c=c):
                    pltpu.sync_copy(rows, agg_ref.at[c].at[pl.ds(base, CHD)])


def _k2(hr, idx_g, dst, w):
    return pl.kernel(
        _k2_body,
        out_type=jax.ShapeDtypeStruct((NC, N, 128), jnp.float32),
        mesh=_MESH,
        compiler_params=_PARAMS,
        scratch_types=[
            pltpu.VMEM((EC,), jnp.int32),
            pltpu.VMEM((NCHK, CH), jnp.int32),
            pltpu.VMEM((EC,), jnp.float32),
            pltpu.VMEM((CH, 128), jnp.float32),
            pltpu.VMEM_SHARED((N, 128), jnp.float32),
            pltpu.SemaphoreType.DMA,
        ],
    )(hr, idx_g, dst.reshape(NS, NCHK, CH), w)


def _k3_body(a0, a1, s0, s1, h1_ref):
    left = a0[0] + s0[0]
    right = a1[0] + s1[0]
    h = jnp.concatenate([left, right[:, 16:]], axis=1)
    h = jnp.where(h > 0, h, h * jnp.float32(_SLOPE))
    h1_ref[...] = jnp.concatenate(
        [h, jnp.zeros((BN, 256 - D), jnp.float32)], axis=1)


def _k3(agg, selfp):
    return pl.pallas_call(
        _k3_body,
        grid=(NBLK,),
        in_specs=[
            pl.BlockSpec((1, BN, 128), lambda i: (0, i, 0)),
            pl.BlockSpec((1, BN, 128), lambda i: (1, i, 0)),
            pl.BlockSpec((1, BN, 128), lambda i: (0, i, 0)),
            pl.BlockSpec((1, BN, 128), lambda i: (1, i, 0)),
        ],
        out_specs=pl.BlockSpec((BN, 256), lambda i: (i, 0)),
        out_shape=jax.ShapeDtypeStruct((N, 256), jnp.float32),
    )(agg, agg, selfp, selfp)


def _k4_body(dst_ref, et_ref, src_ref, cs_ref, h1_ref, sums_ref,
             Db, Eb, Sb, csb, rowbuf, sums_l, merge_sh, sem):
    cid = lax.axis_index("c")
    sid = lax.axis_index("s")

    def z(i, _):
        def zc(m, _):
            sums_l[i, pl.ds(m * L, L)] = jnp.zeros((L,), jnp.float32)
            return 0
        lax.fori_loop(0, 256 // L, zc, 0)
        return 0
    lax.fori_loop(0, 16, z, 0)

    pltpu.sync_copy(dst_ref.at[pl.ds(sid * EC, EC)], Db)
    pltpu.sync_copy(et_ref.at[pl.ds(sid * EC, EC)], Eb)
    pltpu.sync_copy(src_ref.at[pl.ds(sid * EC, EC)], Sb)
    pltpu.sync_copy(cs_ref, csb)
    cv = csb[0, :]
    c0 = cv[0]
    c1 = cv[1]

    def it(i, _):
        dv = Db[pl.ds(i * L, L)]
        m0 = dv == c0
        m1 = dv == c1
        mm = m0 | m1
        pop = plsc.all_reduce_population_count(mm)

        @pl.when(pop[0] > 0)
        def _():
            pltpu.async_copy(h1_ref.at[Sb.at[pl.ds(i * L, L)]],
                             rowbuf, sem).wait()
            ev = Eb[pl.ds(i * L, L)]
            rowv = jnp.where(m1, R, 0) + ev
            mmi = mm.astype(jnp.int32)

            for l in range(L):
                @pl.when(mmi[l] != 0)
                def _(l=l):
                    rr = rowv[l]
                    for m in range(D // L):
                        s = pl.ds(m * L, L)
                        sums_l[rr, s] = sums_l[rr, s] + rowbuf[l, s]
        return 0
    lax.fori_loop(0, EC // L, it, 0)
    pltpu.sync_copy(sums_l, merge_sh.at[sid])
    plsc.subcore_barrier()

    @pl.when(sid == 0)
    def _():
        for t in range(1, NS):
            pltpu.sync_copy(merge_sh.at[t], rowbuf)

            def acc(i, _):
                def ac(m, _):
                    s = pl.ds(m * L, L)
                    sums_l[i, s] = sums_l[i, s] + rowbuf[i, s]
                    return 0
                lax.fori_loop(0, 256 // L, ac, 0)
                return 0
            lax.fori_loop(0, 2 * R, acc, 0)
        for c in range(NC):
            @pl.when(cid == c)
            def _(c=c):
                pltpu.sync_copy(sums_l, sums_ref.at[c])


def _k4(dst, et, src, cs, h1):
    return pl.kernel(
        _k4_body,
        out_type=jax.ShapeDtypeStruct((NC, 2 * R, 256), jnp.float32),
        mesh=_MESH,
        compiler_params=_PARAMS,
        scratch_types=[
            pltpu.VMEM((EC,), jnp.int32),
            pltpu.VMEM((EC,), jnp.int32),
            pltpu.VMEM((EC,), jnp.int32),
            pltpu.VMEM((1, 16), jnp.int32),
            pltpu.VMEM((L, 256), jnp.float32),
            pltpu.VMEM((2 * R, 256), jnp.float32),
            pltpu.VMEM_SHARED((NS, 2 * R, 256), jnp.float32),
            pltpu.SemaphoreType.DMA,
        ],
    )(dst, et, src, cs, h1)


def _k5_body(sums_ref, cnt_ref, cs_ref, h1_ref, bases_ref, comp_ref,
             root_ref, bias_ref, W1_ref, b1_ref, W2_ref, b2_ref,
             o_ref, hbuf, cbuf, sem):
    c0 = cs_ref[0, 0]
    c1 = cs_ref[0, 1]
    for slot, idx in ((0, c0), (1, c1)):
        cp = pltpu.make_async_copy(h1_ref.at[pl.ds(idx, 1)],
                                   hbuf.at[pl.ds(slot, 1)], sem)
        cp.start()
        cp.wait()
        cp = pltpu.make_async_copy(cnt_ref.at[pl.ds(idx, 1)],
                                   cbuf.at[pl.ds(slot, 1)], sem)
        cp.start()
        cp.wait()

    sums = (sums_ref[0, :, :D] + sums_ref[1, :, :D]) * 0.5
    inv2 = 1.0 / jnp.maximum(cbuf[...], 1.0)
    inv16 = jnp.concatenate([inv2[0], inv2[1]])
    mean = sums * lax.broadcast_in_dim(inv16, (2 * R, D), (0,))

    agg = jnp.zeros((2, D), jnp.float32)
    for r in range(R):
        wr = comp_ref[r, 0] * bases_ref[0]
        for b in range(1, NB):
            wr = wr + comp_ref[r, b] * bases_ref[b]
        mt = jnp.concatenate([mean[r:r + 1], mean[R + r:R + r + 1]])
        agg = agg + jnp.dot(mt, wr, preferred_element_type=jnp.float32)

    h2 = agg + jnp.dot(hbuf[:, :D], root_ref[...],
                       preferred_element_type=jnp.float32) + bias_ref[...]
    hc = h2[0]
    hs = h2[1]
    cat = jnp.concatenate([jnp.abs(hs - hc), hs * hc]).reshape(1, 2 * D)
    hid = jnp.dot(cat, W1_ref[...], preferred_element_type=jnp.float32) \
        + b1_ref[...]
    hid = jnp.where(hid > 0, hid, hid * jnp.float32(_SLOPE))
    o_ref[...] = jnp.dot(hid, W2_ref[...],
                         preferred_element_type=jnp.float32) + b2_ref[...]


def _k5(sums2, cnt2d, cs, h1, bases2, comp2, root2, bias2, W1, b1, W2, b2):
    return pl.pallas_call(
        _k5_body,
        in_specs=[
            pl.BlockSpec((NC, 2 * R, 256), lambda: (0, 0, 0)),
            pl.BlockSpec(memory_space=pl.ANY),
            pl.BlockSpec(memory_space=pltpu.SMEM),
            pl.BlockSpec(memory_space=pl.ANY),
            pl.BlockSpec((NB, D, D), lambda: (0, 0, 0)),
            pl.BlockSpec(memory_space=pltpu.SMEM),
            pl.BlockSpec((D, D), lambda: (0, 0)),
            pl.BlockSpec((1, D), lambda: (0, 0)),
            pl.BlockSpec((2 * D, H1), lambda: (0, 0)),
            pl.BlockSpec((1, H1), lambda: (0, 0)),
            pl.BlockSpec((H1, 1), lambda: (0, 0)),
            pl.BlockSpec((1, 1), lambda: (0, 0)),
        ],
        out_specs=pl.BlockSpec((1, 1), lambda: (0, 0)),
        out_shape=jax.ShapeDtypeStruct((1, 1), jnp.float32),
        scratch_shapes=[
            pltpu.VMEM((2, 256), jnp.float32),
            pltpu.VMEM((2, R), jnp.float32),
            pltpu.SemaphoreType.DMA,
        ],
    )(sums2, cnt2d, cs, h1, bases2, comp2, root2, bias2, W1, b1, W2, b2)


def kernel(x, pos_idx, edge_index, edge_type, pos_emb, bases1, comp1, root1,
           bias1, bases2, comp2, root2, bias2, W1, b1, W2, b2):
    src = edge_index[0]
    dst = edge_index[1]
    et = edge_type

    w, idx_g, cnt = _k0(dst, et, src)

    basesp1 = jnp.stack([bases1[:, :, 0:128], bases1[:, :, 112:240]])
    rootp1 = jnp.stack([root1[:, 0:128], root1[:, 112:240]])
    biasp1 = jnp.stack([bias1[0:128].reshape(1, 128),
                        bias1[112:240].reshape(1, 128)])
    pos3 = pos_idx.reshape(NBLK, 1, BN)
    hr, selfp, cs = _k1(x, pos3, pos_emb, basesp1, comp1, rootp1, biasp1)
    agg = _k2(hr, idx_g, dst, w)
    h1 = _k3(agg, selfp)

    sums2 = _k4(dst, et, src, cs, h1)
    out = _k5(sums2, cnt.reshape(N, R), cs, h1, bases2, comp2, root2,
              bias2.reshape(1, D), W1, b1.reshape(1, H1), W2,
              b2.reshape(1, 1))
    return out.reshape(1)
